# baseline (speedup 1.0000x reference)
import functools

import jax
import jax.numpy as jnp
import numpy as np
from jax import lax
from jax.experimental import pallas as pl
from jax.experimental.pallas import tpu as pltpu
from jax.experimental.pallas import tpu_sc as plsc

_V, _D, _NF = 100000, 16, 26
_B, _ND = 1024, 13
_F = 16 + _NF * _D
_TILE = 16
_NT = _F // _TILE
_WU = 512
_NW = 32
_CHUNK = (_B * _NF) // _NW


def _off(i):
    return i * _F - (i * (i - 1)) // 2


def _sc_gather_body(sp_hbm, tab_hbm, out_hbm, idx_v, rows_v, sem):
    wid = lax.axis_index("s") * 2 + lax.axis_index("c")
    base = wid * _CHUNK
    pltpu.sync_copy(sp_hbm.at[pl.ds(base, _CHUNK)], idx_v)
    iota = lax.iota(jnp.int32, 16)
    for k in range(_CHUNK // 16):
        sl = pl.ds(k * 16, 16)
        pos = base + (k * 16) + iota
        fld = lax.rem(pos, _NF)
        idx_v[sl] = idx_v[sl] + fld * _V
    pltpu.async_copy(tab_hbm.at[idx_v], rows_v, sem).wait()
    pltpu.sync_copy(rows_v, out_hbm.at[pl.ds(base, _CHUNK)])


def _sc_gather(sparse_flat, tables_flat):
    return pl.kernel(
        _sc_gather_body,
        out_type=jax.ShapeDtypeStruct((_B * _NF, _D), jnp.float32),
        mesh=plsc.VectorSubcoreMesh(
            core_axis_name="c", subcore_axis_name="s",
            num_cores=2, num_subcores=16),
        scratch_types=[
            pltpu.VMEM((_CHUNK,), jnp.int32),
            pltpu.VMEM((_CHUNK, _D), jnp.float32),
            pltpu.SemaphoreType.DMA,
        ],
        compiler_params=pltpu.CompilerParams(use_tc_tiling_on_sc=False),
    )(sparse_flat, tables_flat)


def _bottom_body(d_ref, w0, b0, w1, b1, w2, b2, out_ref):
    h = jnp.maximum(jnp.dot(d_ref[...], w0[...],
                            preferred_element_type=jnp.float32) + b0[...], 0.0)
    h = jnp.maximum(jnp.dot(h, w1[...],
                            preferred_element_type=jnp.float32) + b1[...], 0.0)
    out_ref[...] = jnp.maximum(
        jnp.dot(h, w2[...], preferred_element_type=jnp.float32) + b2[...], 0.0)


def _bottom(dense_pad, w0p, b0, w1, b1, w2, b2):
    return pl.pallas_call(
        _bottom_body,
        out_shape=jax.ShapeDtypeStruct((_B, 16), jnp.float32),
    )(dense_pad, w0p, b0, w1, b1, w2, b2)


def _main_body(xpf_ref, xpb_ref, xcols_ref, wpk0_ref, wpk_ref, tw0h_ref,
               tb0_ref, tw1_ref, tb1_ref, tw2_ref, tb2_ref, out_ref, acc_ref):
    t = pl.program_id(0)

    @pl.when(t == 0)
    def _tile0_f32():
        xf = xpf_ref[...]
        a = jnp.dot(xf[:, :16], tw0h_ref[...],
                    preferred_element_type=jnp.float32)
        for il in range(_TILE):
            slab = xf[:, il:il + 1] * xf
            a += jnp.dot(slab, wpk0_ref[il * _WU:(il + 1) * _WU, :],
                         preferred_element_type=jnp.float32)
        acc_ref[...] = a

    @pl.when(t > 0)
    def _tiles_bf16():
        xall = xpb_ref[...]
        xI = xcols_ref[0]
        slabs = [xI[:, il:il + 1] * xall for il in range(_TILE)]
        u = jnp.concatenate(slabs, axis=1)
        acc_ref[...] += jnp.dot(u, wpk_ref[...],
                                preferred_element_type=jnp.float32)

    @pl.when(t == _NT - 1)
    def _epilogue():
        z = jnp.maximum(acc_ref[...] + tb0_ref[...], 0.0)
        z = jnp.maximum(
            jnp.dot(z, tw1_ref[...],
                    preferred_element_type=jnp.float32) + tb1_ref[...], 0.0)
        out_ref[...] = jnp.dot(
            z, tw2_ref[...],
            preferred_element_type=jnp.float32) + tb2_ref[...]


def _main(xpf, xpb, xcols, wpk0, wpk, tw0h, tb0, tw1, tb1, tw2, tb2):
    return pl.pallas_call(
        _main_body,
        grid=(_NT,),
        in_specs=[
            pl.BlockSpec((_B, _WU), lambda t: (0, 0)),
            pl.BlockSpec((_B, _WU), lambda t: (0, 0)),
            pl.BlockSpec((1, _B, _TILE), lambda t: (t, 0, 0)),
            pl.BlockSpec((_TILE * _WU, 128), lambda t: (0, 0)),
            pl.BlockSpec((_TILE * _WU, 128),
                         lambda t: (jnp.maximum(t - 1, 0), 0)),
            pl.BlockSpec((16, 128), lambda t: (0, 0)),
            pl.BlockSpec((1, 128), lambda t: (0, 0)),
            pl.BlockSpec((128, 64), lambda t: (0, 0)),
            pl.BlockSpec((1, 64), lambda t: (0, 0)),
            pl.BlockSpec((64, 1), lambda t: (0, 0)),
            pl.BlockSpec((1, 1), lambda t: (0, 0)),
        ],
        out_specs=pl.BlockSpec((_B, 1), lambda t: (0, 0)),
        out_shape=jax.ShapeDtypeStruct((_B, 1), jnp.float32),
        scratch_shapes=[pltpu.VMEM((_B, 128), jnp.float32)],
        compiler_params=pltpu.CompilerParams(
            dimension_semantics=("arbitrary",)),
    )(xpf, xpb, xcols, wpk0, wpk, tw0h, tb0, tw1, tb1, tw2, tb2)


def _pack(w, t_lo, t_hi):
    parts = []
    for t in range(t_lo, t_hi):
        for il in range(_TILE):
            i = _TILE * t + il
            rows = w[16 + _off(i):16 + _off(i) + (_F - i)]
            parts.append(jnp.pad(rows, ((i, _WU - _F), (0, 0))))
    return jnp.concatenate(parts, axis=0)


def kernel(dense_features, sparse_features, emb_tables,
           bw0, bb0, bw1, bb1, bw2, bb2,
           tw0, tb0, tw1, tb1, tw2, tb2):
    emb_rows = _sc_gather(sparse_features.reshape(-1),
                          emb_tables.reshape(_NF * _V, _D))
    emb = emb_rows.reshape(_B, _NF * _D)

    dense_pad = jnp.pad(dense_features, ((0, 0), (0, 3)))
    bw0p = jnp.pad(bw0, ((0, 3), (0, 0)))
    d_out = _bottom(dense_pad, bw0p, bb0.reshape(1, -1),
                    bw1, bb1.reshape(1, -1), bw2, bb2.reshape(1, -1))

    xf = jnp.concatenate([d_out, emb], axis=1)
    xpf = jnp.pad(xf, ((0, 0), (0, _WU - _F)))
    xb = xf.astype(jnp.bfloat16)
    xpb = jnp.pad(xb, ((0, 0), (0, _WU - _F)))
    xcols = xb.reshape(_B, _NT, _TILE).transpose(1, 0, 2)

    wpk0 = _pack(tw0, 0, 1)
    wpk = _pack(tw0.astype(jnp.bfloat16), 1, _NT)
    out = _main(xpf, xpb, xcols, wpk0, wpk,
                tw0[:16], tb0.reshape(1, -1),
                tw1, tb1.reshape(1, -1),
                tw2, tb2.reshape(1, -1))
    return out


# speedup vs baseline: 1.8056x; 1.8056x over previous
import functools

import jax
import jax.numpy as jnp
import numpy as np
from jax import lax
from jax.experimental import pallas as pl
from jax.experimental.pallas import tpu as pltpu
from jax.experimental.pallas import tpu_sc as plsc

_V, _D, _NF = 100000, 16, 26
_B, _ND = 1024, 13
_F = 16 + _NF * _D
_TILE = 16
_NT = _F // _TILE
_WU = 512
_NW = 32
_CHUNK = (_B * _NF) // _NW


def _off(i):
    return i * _F - (i * (i - 1)) // 2


_HALF = _CHUNK // 4


def _sc_gather_body(sp_hbm, tab_hbm, out_hbm, sp_v, idxw, rof_v, blk_v,
                    rows_v, sem):
    wid = lax.axis_index("s") * 2 + lax.axis_index("c")
    base = wid * _CHUNK
    pltpu.sync_copy(sp_hbm.at[pl.ds(base, _CHUNK)], sp_v)
    iota = lax.iota(jnp.int32, 16)
    for h in range(4):
        for k in range(_HALF // 16):
            sl = pl.ds(k * 16, 16)
            gsl = pl.ds(h * _HALF + k * 16, 16)
            pos = base + h * _HALF + k * 16 + iota
            g = sp_v[gsl] + lax.rem(pos, _NF) * _V
            idxw[sl] = lax.shift_right_logical(g, 3)
            rof_v[sl] = lax.rem(g, 8) * _D
        pltpu.async_copy(tab_hbm.at[idxw], blk_v, sem).wait()
        for k in range(_HALF // 16):
            sl = pl.ds(k * 16, 16)
            e = k * 16 + iota
            rof = rof_v[sl]
            for c in range(_D):
                w = plsc.load_gather(blk_v, [e, rof + c])
                plsc.store_scatter(rows_v, [h * _HALF + e, iota * 0 + c], w)
    pltpu.sync_copy(rows_v, out_hbm.at[pl.ds(base, _CHUNK)])


def _sc_gather(sparse_flat, tables_lines):
    return pl.kernel(
        _sc_gather_body,
        out_type=jax.ShapeDtypeStruct((_B * _NF, _D), jnp.float32),
        mesh=plsc.VectorSubcoreMesh(
            core_axis_name="c", subcore_axis_name="s",
            num_cores=2, num_subcores=16),
        scratch_types=[
            pltpu.VMEM((_CHUNK,), jnp.int32),
            pltpu.VMEM((_HALF,), jnp.int32),
            pltpu.VMEM((_HALF,), jnp.int32),
            pltpu.VMEM((_HALF, 128), jnp.float32),
            pltpu.VMEM((_CHUNK, _D), jnp.float32),
            pltpu.SemaphoreType.DMA,
        ],
        compiler_params=pltpu.CompilerParams(
            needs_layout_passes=False, use_tc_tiling_on_sc=False),
    )(sparse_flat, tables_lines)


def _bottom_body(d_ref, w0, b0, w1, b1, w2, b2, out_ref):
    h = jnp.maximum(jnp.dot(d_ref[...], w0[...],
                            preferred_element_type=jnp.float32) + b0[...], 0.0)
    h = jnp.maximum(jnp.dot(h, w1[...],
                            preferred_element_type=jnp.float32) + b1[...], 0.0)
    out_ref[...] = jnp.maximum(
        jnp.dot(h, w2[...], preferred_element_type=jnp.float32) + b2[...], 0.0)


def _bottom(dense_pad, w0p, b0, w1, b1, w2, b2):
    return pl.pallas_call(
        _bottom_body,
        out_shape=jax.ShapeDtypeStruct((_B, 16), jnp.float32),
    )(dense_pad, w0p, b0, w1, b1, w2, b2)


def _main_body(xpf_ref, xpb_ref, xcols_ref, wpk0_ref, wpk_ref, tw0h_ref,
               tb0_ref, tw1_ref, tb1_ref, tw2_ref, tb2_ref, out_ref, acc_ref):
    t = pl.program_id(0)

    @pl.when(t == 0)
    def _tile0_f32():
        xf = xpf_ref[...]
        a = jnp.dot(xf[:, :16], tw0h_ref[...],
                    preferred_element_type=jnp.float32)
        for il in range(_TILE):
            slab = xf[:, il:il + 1] * xf
            a += jnp.dot(slab, wpk0_ref[il * _WU:(il + 1) * _WU, :],
                         preferred_element_type=jnp.float32)
        acc_ref[...] = a

    @pl.when(t > 0)
    def _tiles_bf16():
        xall = xpb_ref[...]
        xI = xcols_ref[0]
        slabs = [xI[:, il:il + 1] * xall for il in range(_TILE)]
        u = jnp.concatenate(slabs, axis=1)
        acc_ref[...] += jnp.dot(u, wpk_ref[...],
                                preferred_element_type=jnp.float32)

    @pl.when(t == _NT - 1)
    def _epilogue():
        z = jnp.maximum(acc_ref[...] + tb0_ref[...], 0.0)
        z = jnp.maximum(
            jnp.dot(z, tw1_ref[...],
                    preferred_element_type=jnp.float32) + tb1_ref[...], 0.0)
        out_ref[...] = jnp.dot(
            z, tw2_ref[...],
            preferred_element_type=jnp.float32) + tb2_ref[...]


def _main(xpf, xpb, xcols, wpk0, wpk, tw0h, tb0, tw1, tb1, tw2, tb2):
    return pl.pallas_call(
        _main_body,
        grid=(_NT,),
        in_specs=[
            pl.BlockSpec((_B, _WU), lambda t: (0, 0)),
            pl.BlockSpec((_B, _WU), lambda t: (0, 0)),
            pl.BlockSpec((1, _B, _TILE), lambda t: (t, 0, 0)),
            pl.BlockSpec((_TILE * _WU, 128), lambda t: (0, 0)),
            pl.BlockSpec((_TILE * _WU, 128),
                         lambda t: (jnp.maximum(t - 1, 0), 0)),
            pl.BlockSpec((16, 128), lambda t: (0, 0)),
            pl.BlockSpec((1, 128), lambda t: (0, 0)),
            pl.BlockSpec((128, 64), lambda t: (0, 0)),
            pl.BlockSpec((1, 64), lambda t: (0, 0)),
            pl.BlockSpec((64, 1), lambda t: (0, 0)),
            pl.BlockSpec((1, 1), lambda t: (0, 0)),
        ],
        out_specs=pl.BlockSpec((_B, 1), lambda t: (0, 0)),
        out_shape=jax.ShapeDtypeStruct((_B, 1), jnp.float32),
        scratch_shapes=[pltpu.VMEM((_B, 128), jnp.float32)],
        compiler_params=pltpu.CompilerParams(
            dimension_semantics=("arbitrary",)),
    )(xpf, xpb, xcols, wpk0, wpk, tw0h, tb0, tw1, tb1, tw2, tb2)


def _pack_window(ref, i, dtype):
    rows = ref[pl.ds(16 + _off(i), _F - i), :]
    parts = [rows, jnp.zeros((_WU - _F, 128), dtype)]
    if i:
        parts.insert(0, jnp.zeros((i, 128), dtype))
    return jnp.concatenate(parts, axis=0)


def _pack_body(tw0f_ref, tw0b_ref, wpk0_ref, wpk_any, sbuf, sem):
    for il in range(_TILE):
        wpk0_ref[il * _WU:(il + 1) * _WU, :] = _pack_window(
            tw0f_ref, il, jnp.float32)
    def out_dma(t, slot):
        return pltpu.make_async_copy(
            sbuf.at[slot], wpk_any.at[pl.ds((t - 1) * _TILE * _WU,
                                            _TILE * _WU), :], sem.at[slot])

    for t in range(1, _NT):
        slot = t % 2
        if t > 2:
            out_dma(t - 2, slot).wait()
        for il in range(_TILE):
            i = _TILE * t + il
            sbuf[slot, il * _WU:(il + 1) * _WU, :] = _pack_window(
                tw0b_ref, i, jnp.bfloat16)
        out_dma(t, slot).start()
    out_dma(_NT - 2, (_NT - 2) % 2).wait()
    out_dma(_NT - 1, (_NT - 1) % 2).wait()


def _pack(tw0f, tw0b):
    vspec = pl.BlockSpec(memory_space=pltpu.MemorySpace.VMEM)
    aspec = pl.BlockSpec(memory_space=pltpu.MemorySpace.HBM)
    return pl.pallas_call(
        _pack_body,
        in_specs=[vspec, vspec],
        out_specs=[vspec, aspec],
        out_shape=[
            jax.ShapeDtypeStruct((_TILE * _WU, 128), jnp.float32),
            jax.ShapeDtypeStruct(((_NT - 1) * _TILE * _WU, 128),
                                 jnp.bfloat16),
        ],
        scratch_shapes=[
            pltpu.VMEM((2, _TILE * _WU, 128), jnp.bfloat16),
            pltpu.SemaphoreType.DMA((2,)),
        ],
    )(tw0f, tw0b)


def kernel(dense_features, sparse_features, emb_tables,
           bw0, bb0, bw1, bb1, bw2, bb2,
           tw0, tb0, tw1, tb1, tw2, tb2):
    emb_rows = _sc_gather(sparse_features.reshape(-1),
                          emb_tables.reshape(_NF * _V // 8, 8 * _D))
    emb = emb_rows.reshape(_B, _NF * _D)

    dense_pad = jnp.pad(dense_features, ((0, 0), (0, 3)))
    bw0p = jnp.pad(bw0, ((0, 3), (0, 0)))
    d_out = _bottom(dense_pad, bw0p, bb0.reshape(1, -1),
                    bw1, bb1.reshape(1, -1), bw2, bb2.reshape(1, -1))

    xf = jnp.concatenate([d_out, emb], axis=1)
    xpf = jnp.pad(xf, ((0, 0), (0, _WU - _F)))
    xb = xf.astype(jnp.bfloat16)
    xpb = jnp.pad(xb, ((0, 0), (0, _WU - _F)))
    xcols = xb.reshape(_B, _NT, _TILE).transpose(1, 0, 2)

    tw0f = tw0[:16 + _off(15) + _WU]
    wpk0, wpk = _pack(tw0f, tw0.astype(jnp.bfloat16))
    out = _main(xpf, xpb, xcols, wpk0, wpk,
                tw0[:16], tb0.reshape(1, -1),
                tw1, tb1.reshape(1, -1),
                tw2, tb2.reshape(1, -1))
    return out
